# baseline (speedup 1.0000x reference)
import functools

import jax
import jax.numpy as jnp
from jax import lax
from jax.experimental import pallas as pl
from jax.experimental.pallas import tpu as pltpu
from jax.experimental.pallas import tpu_sc as plsc

F32 = jnp.float32
NC = 2
NS = 16
NW = NC * NS
CH = 128
K = 6
BLK = K * CH


def _round_up(a, m):
    return (a + m - 1) // m * m


@functools.partial(jax.jit, static_argnames=("np_", "ep", "w"))
def _sc_deg(dst2, ones_hbm, zeros_hbm, zrows_hbm, *, np_, ep, w):
    nb = ep // BLK
    rpt = np_ // NS
    mesh = plsc.VectorSubcoreMesh(core_axis_name="c", subcore_axis_name="s")

    @functools.partial(
        pl.kernel,
        out_type=[jax.ShapeDtypeStruct((np_, w), F32),
                  jax.ShapeDtypeStruct((np_, w), F32)],
        mesh=mesh,
        scratch_types=[
            pltpu.VMEM((2, K, CH), jnp.int32),
            pltpu.VMEM((CH, w), F32),
            pltpu.VMEM_SHARED((np_, w), F32),
            pltpu.SemaphoreType.DMA,
        ],
        compiler_params=pltpu.CompilerParams(use_tc_tiling_on_sc=False),
    )
    def k(dst_hbm, ones_h, zeros_h, zrows_h, out0, out1, didx, ones_v, acc, ssem):
        c = lax.axis_index("c")
        s = lax.axis_index("s")
        tid = c * NS + s
        r0 = tid * (ep // CH)
        pltpu.sync_copy(ones_h, ones_v)
        pltpu.sync_copy(zeros_h, acc.at[pl.ds(s * rpt, rpt)])
        pltpu.sync_copy(dst_hbm.at[pl.ds(r0, K)], didx.at[0])
        plsc.subcore_barrier()

        def step(b, slot, nslot, pred):
            for j in range(K):
                pltpu.async_copy(ones_v, acc.at[didx.at[slot, j]], ssem, add=True)
            @pl.when(pred)
            def _():
                pltpu.sync_copy(dst_hbm.at[pl.ds(r0 + (b + 1) * K, K)],
                                didx.at[nslot])
            for j in range(K):
                pltpu.make_async_copy(zrows_h.at[0], ones_v, ssem).wait()

        def body(bp, carry):
            b0 = 2 * bp
            step(b0, 0, 1, b0 + 1 < nb)
            step(b0 + 1, 1, 0, b0 + 2 < nb)
            return carry

        lax.fori_loop(0, nb // 2, body, 0)
        plsc.subcore_barrier()

        @pl.when(c == 0)
        def _():
            pltpu.sync_copy(acc.at[pl.ds(s * rpt, rpt)], out0.at[pl.ds(s * rpt, rpt)])

        @pl.when(c == 1)
        def _():
            pltpu.sync_copy(acc.at[pl.ds(s * rpt, rpt)], out1.at[pl.ds(s * rpt, rpt)])

    return k(dst2, ones_hbm, zeros_hbm, zrows_hbm)


@functools.partial(jax.jit, static_argnames=("np_", "ep", "w"))
def _sc_spmm(table, src2, dst2, zeros_hbm, zrows_hbm, *, np_, ep, w):
    nb = ep // BLK
    rpt = np_ // NS
    mesh = plsc.VectorSubcoreMesh(core_axis_name="c", subcore_axis_name="s")

    @functools.partial(
        pl.kernel,
        out_type=[jax.ShapeDtypeStruct((np_, w), F32),
                  jax.ShapeDtypeStruct((np_, w), F32)],
        mesh=mesh,
        scratch_types=[
            pltpu.VMEM((2, K, CH), jnp.int32),
            pltpu.VMEM((2, K, CH), jnp.int32),
            pltpu.VMEM((2, K, CH, w), F32),
            pltpu.VMEM_SHARED((np_, w), F32),
            pltpu.VMEM_SHARED((np_, w), F32),
            pltpu.SemaphoreType.DMA,
            pltpu.SemaphoreType.DMA,
        ],
        compiler_params=pltpu.CompilerParams(use_tc_tiling_on_sc=False),
    )
    def k(tab_hbm, src_hbm, dst_hbm, zeros_h, zrows_h, out0, out1,
          sidx, didx, rows, acc, tabs, gsem, ssem):
        c = lax.axis_index("c")
        s = lax.axis_index("s")
        tid = c * NS + s
        r0 = tid * (ep // CH)
        pltpu.sync_copy(zeros_h, acc.at[pl.ds(s * rpt, rpt)])
        pltpu.sync_copy(tab_hbm.at[pl.ds(s * rpt, rpt)],
                        tabs.at[pl.ds(s * rpt, rpt)])
        pltpu.sync_copy(src_hbm.at[pl.ds(r0, K)], sidx.at[0])
        pltpu.sync_copy(dst_hbm.at[pl.ds(r0, K)], didx.at[0])
        plsc.subcore_barrier()
        for j in range(K):
            pltpu.async_copy(tabs.at[sidx.at[0, j]], rows.at[0, j], gsem)

        def step(b, slot, nslot, pred):
            pltpu.make_async_copy(zrows_h, rows.at[slot], gsem).wait()
            for j in range(K):
                pltpu.async_copy(rows.at[slot, j], acc.at[didx.at[slot, j]],
                                 ssem, add=True)
            @pl.when(pred)
            def _():
                pltpu.sync_copy(src_hbm.at[pl.ds(r0 + (b + 1) * K, K)],
                                sidx.at[nslot])
                pltpu.sync_copy(dst_hbm.at[pl.ds(r0 + (b + 1) * K, K)],
                                didx.at[nslot])
                for j in range(K):
                    pltpu.async_copy(tabs.at[sidx.at[nslot, j]],
                                     rows.at[nslot, j], gsem)
            pltpu.make_async_copy(zrows_h, rows.at[slot], ssem).wait()

        def body(bp, carry):
            b0 = 2 * bp
            step(b0, 0, 1, b0 + 1 < nb)
            step(b0 + 1, 1, 0, b0 + 2 < nb)
            return carry

        lax.fori_loop(0, nb // 2, body, 0)
        plsc.subcore_barrier()

        @pl.when(c == 0)
        def _():
            pltpu.sync_copy(acc.at[pl.ds(s * rpt, rpt)], out0.at[pl.ds(s * rpt, rpt)])

        @pl.when(c == 1)
        def _():
            pltpu.sync_copy(acc.at[pl.ds(s * rpt, rpt)], out1.at[pl.ds(s * rpt, rpt)])

    return k(table, src2, dst2, zeros_hbm, zrows_hbm)


def _flat_call(body, flat_args, mats, n_out):
    shp = flat_args[0].shape
    out_shape = [jax.ShapeDtypeStruct(shp, F32) for _ in range(n_out)]
    return pl.pallas_call(
        body,
        out_shape=out_shape if n_out > 1 else out_shape[0],
    )(*flat_args, *mats)


def _tc_a(daf, dbf, xf):
    def body(da, db, xr, dinv_o, xs1_o):
        dinv = lax.rsqrt(1.0 + da[...] + db[...])
        dinv_o[...] = dinv
        xs1_o[...] = xr[...] * dinv

    return _flat_call(body, [daf, dbf, xf], [], 2)


def _tc_b(dinvf, xs1f, s1af, s1bf, BD1, b1t):
    def body(dv, x1, sa, sb, bd, bb, o):
        ax = dv[...] * (sa[...] + sb[...] + x1[...])
        h = jnp.dot(ax, bd[...], preferred_element_type=F32)
        h = jnp.maximum(h + bb[...], 0.0)
        o[...] = dv[...] * h

    return _flat_call(body, [dinvf, xs1f, s1af, s1bf], [BD1, b1t], 1)


def _tc_c(dinvf, xs2f, s2af, s2bf, B0, B1, C0, C1, b2t):
    def body(dv, x2, sa, sb, b0, b1_, c0, c1, bb, o):
        ah = dv[...] * (sa[...] + sb[...] + x2[...])
        e0 = jnp.maximum(jnp.dot(ah, b0[...], preferred_element_type=F32)
                         + bb[...], 0.0)
        e1 = jnp.maximum(jnp.dot(ah, b1_[...], preferred_element_type=F32)
                         + bb[...], 0.0)
        g = (jnp.dot(e0, c0[...], preferred_element_type=F32)
             + jnp.dot(e1, c1[...], preferred_element_type=F32))
        o[...] = dv[...] * g

    return _flat_call(body, [dinvf, xs2f, s2af, s2bf], [B0, B1, C0, C1, b2t], 1)


def _tc_d(dinvf, xs3f, s3af, s3bf, b3t):
    def body(dv, x3, sa, sb, bb, o):
        o[...] = dv[...] * (sa[...] + sb[...] + x3[...]) + bb[...]

    return _flat_call(body, [dinvf, xs3f, s3af, s3bf], [b3t], 1)


def kernel(x, edge_index, W1, b1, W2, b2, W3, b3):
    n = x.shape[0]
    e = edge_index.shape[1]
    np_ = _round_up(n, NS * 8)
    ep = _round_up(-(-e // NW), 2 * BLK)
    e2 = ep * NW
    pad = e2 - e

    pad_ar = jnp.arange(pad, dtype=jnp.int32)
    pad_dst = n + pad_ar % (np_ - n)
    pad_src = (pad_ar * 127) % n
    src2 = jnp.concatenate([edge_index[0], pad_src]).reshape(-1, CH)
    dst2 = jnp.concatenate([edge_index[1], pad_dst]).reshape(-1, CH)

    rpt = np_ // NS
    ones = jnp.ones((CH, 16), F32)
    zerosw = jnp.zeros((rpt, 16), F32)
    zrows = jnp.zeros((K, CH, 16), F32)
    x_p = jnp.pad(x, ((0, np_ - n), (0, 16 - x.shape[1])))

    fshape = (np_ * 16 // 128, 128)
    flat = lambda a: a.reshape(fshape)
    tab = lambda f: f.reshape(np_, 16)

    W1p = jnp.pad(W1, ((0, 16 - W1.shape[0]), (0, 0)))
    BD1 = jnp.kron(jnp.eye(8, dtype=F32), W1p)
    S0 = jnp.eye(8, 4, dtype=F32)
    S1 = jnp.eye(8, 4, k=-4, dtype=F32)
    B0 = jnp.kron(S0, W2)
    B1 = jnp.kron(S1, W2)
    C0 = jnp.kron(S0.T, W3)
    C1 = jnp.kron(S1.T, W3)
    b1t = jnp.tile(b1, 8).reshape(1, 128)
    b2t = jnp.tile(b2, 4).reshape(1, 128)
    b3t = jnp.tile(b3, 8).reshape(1, 128)

    dega, degb = _sc_deg(dst2, ones, zerosw, zrows, np_=np_, ep=ep, w=16)
    dinvf, xs1f = _tc_a(flat(dega), flat(degb), flat(x_p))
    s1a, s1b = _sc_spmm(tab(xs1f), src2, dst2, zerosw, zrows,
                        np_=np_, ep=ep, w=16)
    xs2f = _tc_b(dinvf, xs1f, flat(s1a), flat(s1b), BD1, b1t)
    s2a, s2b = _sc_spmm(tab(xs2f), src2, dst2, zerosw, zrows,
                        np_=np_, ep=ep, w=16)
    xs3f = _tc_c(dinvf, xs2f, flat(s2a), flat(s2b), B0, B1, C0, C1, b2t)
    s3a, s3b = _sc_spmm(tab(xs3f), src2, dst2, zerosw, zrows,
                        np_=np_, ep=ep, w=16)
    outf = _tc_d(dinvf, xs3f, flat(s3a), flat(s3b), b3t)
    return outf.reshape(np_, 16)[:n]


# speedup vs baseline: 1.0734x; 1.0734x over previous
import functools

import jax
import jax.numpy as jnp
from jax import lax
from jax.experimental import pallas as pl
from jax.experimental.pallas import tpu as pltpu
from jax.experimental.pallas import tpu_sc as plsc

F32 = jnp.float32
NC = 2
NS = 16
NW = NC * NS
CH = 128
K = 16
BLK = K * CH


def _round_up(a, m):
    return (a + m - 1) // m * m


@functools.partial(jax.jit, static_argnames=("np_", "ep", "w"))
def _sc_deg(dst2, ones_hbm, zeros_hbm, zrows_hbm, *, np_, ep, w):
    nb = ep // BLK
    rpt = np_ // NS
    mesh = plsc.VectorSubcoreMesh(core_axis_name="c", subcore_axis_name="s")

    @functools.partial(
        pl.kernel,
        out_type=[jax.ShapeDtypeStruct((np_, w), F32),
                  jax.ShapeDtypeStruct((np_, w), F32)],
        mesh=mesh,
        scratch_types=[
            pltpu.VMEM((2, K, CH), jnp.int32),
            pltpu.VMEM((CH, w), F32),
            pltpu.VMEM_SHARED((np_, w), F32),
            pltpu.SemaphoreType.DMA,
        ],
        compiler_params=pltpu.CompilerParams(use_tc_tiling_on_sc=False),
    )
    def k(dst_hbm, ones_h, zeros_h, zrows_h, out0, out1, didx, ones_v, acc, ssem):
        c = lax.axis_index("c")
        s = lax.axis_index("s")
        tid = c * NS + s
        r0 = tid * (ep // CH)
        pltpu.sync_copy(ones_h, ones_v)
        pltpu.sync_copy(zeros_h, acc.at[pl.ds(s * rpt, rpt)])
        pltpu.sync_copy(dst_hbm.at[pl.ds(r0, K)], didx.at[0])
        plsc.subcore_barrier()

        def step(b, slot, nslot, pred):
            for j in range(K):
                pltpu.async_copy(ones_v, acc.at[didx.at[slot, j]], ssem, add=True)
            @pl.when(pred)
            def _():
                pltpu.sync_copy(dst_hbm.at[pl.ds(r0 + (b + 1) * K, K)],
                                didx.at[nslot])
            for j in range(K):
                pltpu.make_async_copy(zrows_h.at[0], ones_v, ssem).wait()

        def body(bp, carry):
            b0 = 2 * bp
            step(b0, 0, 1, b0 + 1 < nb)
            step(b0 + 1, 1, 0, b0 + 2 < nb)
            return carry

        lax.fori_loop(0, nb // 2, body, 0)
        plsc.subcore_barrier()

        @pl.when(c == 0)
        def _():
            pltpu.sync_copy(acc.at[pl.ds(s * rpt, rpt)], out0.at[pl.ds(s * rpt, rpt)])

        @pl.when(c == 1)
        def _():
            pltpu.sync_copy(acc.at[pl.ds(s * rpt, rpt)], out1.at[pl.ds(s * rpt, rpt)])

    return k(dst2, ones_hbm, zeros_hbm, zrows_hbm)


@functools.partial(jax.jit, static_argnames=("np_", "ep", "w"))
def _sc_spmm(table, src2, dst2, zeros_hbm, zrows_hbm, *, np_, ep, w):
    nb = ep // BLK
    rpt = np_ // NS
    mesh = plsc.VectorSubcoreMesh(core_axis_name="c", subcore_axis_name="s")

    @functools.partial(
        pl.kernel,
        out_type=[jax.ShapeDtypeStruct((np_, w), F32),
                  jax.ShapeDtypeStruct((np_, w), F32)],
        mesh=mesh,
        scratch_types=[
            pltpu.VMEM((2, K, CH), jnp.int32),
            pltpu.VMEM((2, K, CH), jnp.int32),
            pltpu.VMEM((2, K, CH, w), F32),
            pltpu.VMEM_SHARED((np_, w), F32),
            pltpu.SemaphoreType.DMA,
            pltpu.SemaphoreType.DMA,
        ],
        compiler_params=pltpu.CompilerParams(use_tc_tiling_on_sc=False),
    )
    def k(tab_hbm, src_hbm, dst_hbm, zeros_h, zrows_h, out0, out1,
          sidx, didx, rows, acc, gsem, ssem):
        c = lax.axis_index("c")
        s = lax.axis_index("s")
        tid = c * NS + s
        r0 = tid * (ep // CH)
        pltpu.sync_copy(zeros_h, acc.at[pl.ds(s * rpt, rpt)])
        pltpu.sync_copy(src_hbm.at[pl.ds(r0, K)], sidx.at[0])
        pltpu.sync_copy(dst_hbm.at[pl.ds(r0, K)], didx.at[0])
        for j in range(K):
            pltpu.async_copy(tab_hbm.at[sidx.at[0, j]], rows.at[0, j], gsem)
        plsc.subcore_barrier()

        def step(b, slot, nslot, pred):
            pltpu.make_async_copy(zrows_h, rows.at[slot], gsem).wait()
            for j in range(K):
                pltpu.async_copy(rows.at[slot, j], acc.at[didx.at[slot, j]],
                                 ssem, add=True)
            @pl.when(pred)
            def _():
                pltpu.sync_copy(src_hbm.at[pl.ds(r0 + (b + 1) * K, K)],
                                sidx.at[nslot])
                pltpu.sync_copy(dst_hbm.at[pl.ds(r0 + (b + 1) * K, K)],
                                didx.at[nslot])
                for j in range(K):
                    pltpu.async_copy(tab_hbm.at[sidx.at[nslot, j]],
                                     rows.at[nslot, j], gsem)
            pltpu.make_async_copy(zrows_h, rows.at[slot], ssem).wait()

        def body(bp, carry):
            b0 = 2 * bp
            step(b0, 0, 1, b0 + 1 < nb)
            step(b0 + 1, 1, 0, b0 + 2 < nb)
            return carry

        lax.fori_loop(0, nb // 2, body, 0)
        plsc.subcore_barrier()

        @pl.when(c == 0)
        def _():
            pltpu.sync_copy(acc.at[pl.ds(s * rpt, rpt)], out0.at[pl.ds(s * rpt, rpt)])

        @pl.when(c == 1)
        def _():
            pltpu.sync_copy(acc.at[pl.ds(s * rpt, rpt)], out1.at[pl.ds(s * rpt, rpt)])

    return k(table, src2, dst2, zeros_hbm, zrows_hbm)


def _flat_call(body, flat_args, mats, n_out):
    shp = flat_args[0].shape
    out_shape = [jax.ShapeDtypeStruct(shp, F32) for _ in range(n_out)]
    return pl.pallas_call(
        body,
        out_shape=out_shape if n_out > 1 else out_shape[0],
    )(*flat_args, *mats)


def _tc_a(daf, dbf, xf):
    def body(da, db, xr, dinv_o, xs1_o):
        dinv = lax.rsqrt(1.0 + da[...] + db[...])
        dinv_o[...] = dinv
        xs1_o[...] = xr[...] * dinv

    return _flat_call(body, [daf, dbf, xf], [], 2)


def _tc_b(dinvf, xs1f, s1af, s1bf, BD1, b1t):
    def body(dv, x1, sa, sb, bd, bb, o):
        ax = dv[...] * (sa[...] + sb[...] + x1[...])
        h = jnp.dot(ax, bd[...], preferred_element_type=F32)
        h = jnp.maximum(h + bb[...], 0.0)
        o[...] = dv[...] * h

    return _flat_call(body, [dinvf, xs1f, s1af, s1bf], [BD1, b1t], 1)


def _tc_c(dinvf, xs2f, s2af, s2bf, B0, B1, C0, C1, b2t):
    def body(dv, x2, sa, sb, b0, b1_, c0, c1, bb, o):
        ah = dv[...] * (sa[...] + sb[...] + x2[...])
        e0 = jnp.maximum(jnp.dot(ah, b0[...], preferred_element_type=F32)
                         + bb[...], 0.0)
        e1 = jnp.maximum(jnp.dot(ah, b1_[...], preferred_element_type=F32)
                         + bb[...], 0.0)
        g = (jnp.dot(e0, c0[...], preferred_element_type=F32)
             + jnp.dot(e1, c1[...], preferred_element_type=F32))
        o[...] = dv[...] * g

    return _flat_call(body, [dinvf, xs2f, s2af, s2bf], [B0, B1, C0, C1, b2t], 1)


def _tc_d(dinvf, xs3f, s3af, s3bf, b3t):
    def body(dv, x3, sa, sb, bb, o):
        o[...] = dv[...] * (sa[...] + sb[...] + x3[...]) + bb[...]

    return _flat_call(body, [dinvf, xs3f, s3af, s3bf], [b3t], 1)


def kernel(x, edge_index, W1, b1, W2, b2, W3, b3):
    n = x.shape[0]
    e = edge_index.shape[1]
    np_ = _round_up(n, NS * 8)
    ep = _round_up(-(-e // NW), 2 * BLK)
    e2 = ep * NW
    pad = e2 - e

    pad_ar = jnp.arange(pad, dtype=jnp.int32)
    pad_dst = n + pad_ar % (np_ - n)
    pad_src = (pad_ar * 127) % n
    src2 = jnp.concatenate([edge_index[0], pad_src]).reshape(-1, CH)
    dst2 = jnp.concatenate([edge_index[1], pad_dst]).reshape(-1, CH)

    rpt = np_ // NS
    ones = jnp.ones((CH, 16), F32)
    zerosw = jnp.zeros((rpt, 16), F32)
    zrows = jnp.zeros((K, CH, 16), F32)
    x_p = jnp.pad(x, ((0, np_ - n), (0, 16 - x.shape[1])))

    fshape = (np_ * 16 // 128, 128)
    flat = lambda a: a.reshape(fshape)
    tab = lambda f: f.reshape(np_, 16)

    W1p = jnp.pad(W1, ((0, 16 - W1.shape[0]), (0, 0)))
    BD1 = jnp.kron(jnp.eye(8, dtype=F32), W1p)
    S0 = jnp.eye(8, 4, dtype=F32)
    S1 = jnp.eye(8, 4, k=-4, dtype=F32)
    B0 = jnp.kron(S0, W2)
    B1 = jnp.kron(S1, W2)
    C0 = jnp.kron(S0.T, W3)
    C1 = jnp.kron(S1.T, W3)
    b1t = jnp.tile(b1, 8).reshape(1, 128)
    b2t = jnp.tile(b2, 4).reshape(1, 128)
    b3t = jnp.tile(b3, 8).reshape(1, 128)

    dega, degb = _sc_deg(dst2, ones, zerosw, zrows, np_=np_, ep=ep, w=16)
    dinvf, xs1f = _tc_a(flat(dega), flat(degb), flat(x_p))
    s1a, s1b = _sc_spmm(tab(xs1f), src2, dst2, zerosw, zrows,
                        np_=np_, ep=ep, w=16)
    xs2f = _tc_b(dinvf, xs1f, flat(s1a), flat(s1b), BD1, b1t)
    s2a, s2b = _sc_spmm(tab(xs2f), src2, dst2, zerosw, zrows,
                        np_=np_, ep=ep, w=16)
    xs3f = _tc_c(dinvf, xs2f, flat(s2a), flat(s2b), B0, B1, C0, C1, b2t)
    s3a, s3b = _sc_spmm(tab(xs3f), src2, dst2, zerosw, zrows,
                        np_=np_, ep=ep, w=16)
    outf = _tc_d(dinvf, xs3f, flat(s3a), flat(s3b), b3t)
    if (n * 16) % 128 == 0:
        return outf[: n * 16 // 128].reshape(n, 16)
    return outf.reshape(np_, 16)[:n]
